# baseline (speedup 1.0000x reference)
import jax
import jax.numpy as jnp
from jax.experimental import pallas as pl


def _norm_no_nan(x, axis=-1, keepdims=False, eps=1e-8):
    return jnp.sqrt(jnp.maximum(jnp.sum(jnp.square(x), axis=axis, keepdims=keepdims), eps))


def _gvp(p, s, v, scalar_act=True, vector_act=True):
    vt = jnp.swapaxes(v, -1, -2)
    vh = vt @ p['wh']
    vn = _norm_no_nan(vh, axis=-2)
    s = jnp.concatenate([s, vn], axis=-1) @ p['ws'] + p['bs']
    vo = jnp.swapaxes(vh @ p['wv'], -1, -2)
    if vector_act:
        vo = vo * jax.nn.sigmoid(_norm_no_nan(vo, axis=-1, keepdims=True))
    if scalar_act:
        s = jax.nn.relu(s)
    return s, vo


def _seg_softmax(x, index, n):
    xm = jax.ops.segment_max(x, index, num_segments=n)
    x = jnp.exp(x - xm[index])
    den = jax.ops.segment_sum(x, index, num_segments=n)
    return x / (den[index] + 1e-16)


def _flood_layer(p, src, dst, s, v, n):
    s, v = _gvp(p['n_encode'], s, v)
    s_i, s_j = s[dst], s[src]
    v_i, v_j = v[dst], v[src]
    s_att = jax.nn.leaky_relu(jnp.sum(s_i * s_j, axis=1, keepdims=True), 0.2)
    s_att = _seg_softmax(s_att, dst, n)
    v_att = jax.nn.leaky_relu(jnp.sum(v_i * v_j, axis=(-2, -1), keepdims=True), 0.2)
    v_att = _seg_softmax(v_att, dst, n)
    s_m, v_m = _gvp(p['m_gvp'], s_att * s_j, v_att * v_j)
    s_a = jax.ops.segment_sum(s_m, dst, num_segments=n)
    v_a = jax.ops.segment_sum(v_m, dst, num_segments=n)
    return _gvp(p['u_gvp'], s_a, v_a)


def _flood_block(p, src, dst, s, v, s_hid, v_hid, valid, n):
    s, v = _gvp(p['gvp_layer'], s, v, False, False)
    s_gz, v_gz = _flood_layer(p['z_in'], src, dst, s, v, n)
    s_gr, v_gr = _flood_layer(p['r_in'], src, dst, s, v, n)
    if s_hid is not None:
        a, b = _flood_layer(p['z_h'], src, dst, s_hid, v_hid, n)
        s_gz, v_gz = s_gz + a, v_gz + b
        a, b = _flood_layer(p['r_h'], src, dst, s_hid, v_hid, n)
        s_gr, v_gr = s_gr + a, v_gr + b
    s_gz, v_gz = jax.nn.sigmoid(s_gz), jax.nn.sigmoid(v_gz)
    s_gr, v_gr = jax.nn.sigmoid(s_gr), jax.nn.sigmoid(v_gr)
    s_hh, v_hh = _flood_layer(p['hh_in'], src, dst, s, v, n)
    if s_hid is not None:
        a, b = _flood_layer(p['hh_h'], src, dst, s_gr * s_hid, v_gr * v_hid, n)
        s_hh, v_hh = s_hh + a, v_hh + b
    s_hh, v_hh = jnp.tanh(s_hh), jnp.tanh(v_hh)
    s_t, v_t = (1 - s_gz) * s_hh, (1 - v_gz) * v_hh
    if s_hid is not None:
        s_t, v_t = s_gz * s_hid + s_t, v_gz * v_hid + v_t
    mask = valid[:, None]
    s_t = jnp.where(mask, s_t, 0.0)
    v_t = jnp.where(mask[..., None], v_t, 0.0)
    return s_t, v_t


def _feat_pred(p, s, v):
    s, v = _gvp(p['gvp1'], s, v)
    l_h, v_out = _gvp(p['gvp2'], s, v, False, False)
    v_old = v_out
    l_h = l_h ** 2
    s_out = _norm_no_nan(v_out, axis=-1)
    v_out = v_out / s_out[..., None]
    return l_h, (s_out, v_out), v_old


def _compute_regression(p, s, v, valid):
    mask = valid[:, None]
    l_h, (s_h, v_h), v_old = _feat_pred(p, s, v)
    l_h = jnp.where(mask, l_h, 0.0)
    s_h = jnp.where(mask, s_h, 0.0)
    v_h = jnp.where(mask[..., None], v_h, 0.0)
    return l_h, (s_h, v_h), v_old


def _identity_pallas(x):
    def body(x_ref, o_ref):
        o_ref[...] = x_ref[...]
    return pl.pallas_call(
        body, out_shape=jax.ShapeDtypeStruct(x.shape, x.dtype))(x)


def kernel(edge_index, s_static, seq, binary, x_v, x_v_norm, rain, wdfp, batch, params):
    n = s_static.shape[0]
    loops = jnp.arange(n, dtype=edge_index.dtype)
    src = jnp.concatenate([edge_index[0], loops])
    dst = jnp.concatenate([edge_index[1], loops])
    bmask = binary.astype(bool)
    v = jnp.where(bmask[..., None], x_v, 0.0)
    v_norm = jnp.where(bmask, x_v_norm, 0.0)
    s_h, v_h = v_norm[:, 0], v[:, 0]
    l_h = wdfp[:, 0]
    r_h = rain[:, 0]
    seq_len = wdfp.shape[1] - 1
    s_h0, v_h0 = None, None
    label_loss = jnp.zeros((), jnp.float32)
    feat_loss = jnp.zeros((), jnp.float32)
    out_labels = []
    pp = params['processor']
    fp = params['feat_pred']
    for i in range(1, seq_len + 1):
        s_in = jnp.concatenate([s_static, s_h, r_h, l_h], axis=-1)
        valid = seq[:, i - 1]
        s_h0, v_h0 = _flood_block(pp, src, dst, s_in, v_h, s_h0, v_h0, valid, n)
        l_h, (s_h, v_h), v_old = _compute_regression(fp, s_h0, v_h0, valid)
        b_t = binary[:, i]
        use_t = jnp.logical_not(jnp.logical_or(jnp.all(b_t == 0), jnp.all(b_t == 1)))
        m = valid.astype(jnp.float32)[:, None]
        label_loss = label_loss + jnp.where(
            use_t,
            jnp.sum(jnp.abs(l_h.reshape(n, -1) - wdfp[:, i].reshape(n, -1)) * m),
            jnp.zeros((), jnp.float32))
        targ = (v[:, i] * v_norm[:, i][..., None]).reshape(n, -1)
        feat_loss = feat_loss + jnp.where(
            use_t,
            jnp.sum(jnp.abs(v_old.reshape(n, -1) - targ) * m),
            jnp.zeros((), jnp.float32))
        out_labels.append(l_h)
        r_h = rain[:, i]
    loss = label_loss + feat_loss
    out_labels = jnp.stack(out_labels, axis=1).squeeze(-1)
    out_labels = _identity_pallas(out_labels)
    return out_labels, loss, seq


# speedup vs baseline: 13.8843x; 13.8843x over previous
import functools

import jax
import jax.numpy as jnp
from jax import lax
from jax.experimental import pallas as pl
from jax.experimental.pallas import tpu as pltpu
from jax.experimental.pallas import tpu_sc as plsc

N_NODES = 50000
NVW = 64
NN = 782
N_PAD = NVW * NN
E_TOT = 850000
E_PAD = 850048

_F32 = jnp.float32
_I32 = jnp.int32


def _norm_no_nan(x, axis=-1, keepdims=False, eps=1e-8):
    return jnp.sqrt(jnp.maximum(jnp.sum(jnp.square(x), axis=axis, keepdims=keepdims), eps))


def _gvp(p, s, v, scalar_act=True, vector_act=True):
    vt = jnp.swapaxes(v, -1, -2)
    vh = vt @ p['wh']
    vn = _norm_no_nan(vh, axis=-2)
    s = jnp.concatenate([s, vn], axis=-1) @ p['ws'] + p['bs']
    vo = jnp.swapaxes(vh @ p['wv'], -1, -2)
    if vector_act:
        vo = vo * jax.nn.sigmoid(_norm_no_nan(vo, axis=-1, keepdims=True))
    if scalar_act:
        s = jax.nn.relu(s)
    return s, vo


def _edge_body(src_ref, dst_ref, bnd_ref, t1_ref, t2_ref, bs_ref, wsv_ref,
               sa_ref, va_ref, raw_ref,
               bnd_v, bs_v, wsv_v, sa_loc, va_loc,
               mx_s, mx_v, den_s, den_v,
               idxs_v, idxd_v, t1s, t1d, t2b, rawbuf, shf, shi,
               sem1, sem2):
    wid = lax.axis_index("s") * 2 + lax.axis_index("c")
    pltpu.sync_copy(bnd_ref, bnd_v)
    pltpu.sync_copy(bs_ref, bs_v)
    pltpu.sync_copy(wsv_ref, wsv_v)
    iota = lax.iota(_I32, 16)
    zeros16 = jnp.zeros((16,), _F32)
    neg16 = jnp.full((16,), -3.0e38, _F32)
    bs_c = [bs_v[pl.ds(j * 16, 16)] for j in range(4)]
    wsv_c = [[wsv_v[pl.ds(hh * 64 + j * 16, 16)] for j in range(4)]
             for hh in range(8)]

    def seg_masks(dstv):
        shi[pl.ds(16, 16)] = dstv
        shi[pl.ds(32, 16)] = dstv
        masks = tuple((iota >= s) & (shi[pl.ds(16 - s, 16)] == dstv)
                      for s in (1, 2, 4, 8))
        run_end = (shi[pl.ds(17, 16)] != dstv) | (iota == 15)
        return masks, run_end

    def seg_scan(x, masks, op):
        for si, s in enumerate((1, 2, 4, 8)):
            shf[pl.ds(16, 16)] = x
            x = jnp.where(masks[si], op(x, shf[pl.ds(16 - s, 16)]), x)
        return x

    def vw_body(h, _):
        vw = wid * 2 + h
        n0 = vw * NN
        e01 = bnd_v[pl.ds(vw, 16)]
        e0 = e01[0]
        e1 = e01[1]
        sb0 = e0 // 64
        sb1 = (e1 + 63) // 64

        def z_sa(i, _):
            sa_loc[pl.ds(i * 16, 16)] = zeros16
            return 0
        lax.fori_loop(0, NN * 4, z_sa, 0)

        def z_va(i, _):
            va_loc[pl.ds(i * 16, 16)] = zeros16
            return 0
        lax.fori_loop(0, NN, z_va, 0)

        def z_st(i, _):
            mx_s[pl.ds(i * 16, 16)] = neg16
            mx_v[pl.ds(i * 16, 16)] = neg16
            den_s[pl.ds(i * 16, 16)] = zeros16
            den_v[pl.ds(i * 16, 16)] = zeros16
            return 0
        lax.fori_loop(0, 49, z_st, 0)

        def lane_ctx(base, rofs):
            dstv = idxd_v[pl.ds(rofs, 16)]
            eidx = base + rofs + iota
            valid = (eidx >= e0) & (eidx < e1)
            dstl = jnp.clip(dstv - n0, 0, NN - 1)
            return dstv, valid, dstl

        def passA(sb, _):
            base = sb * 64
            pltpu.sync_copy(src_ref.at[pl.ds(base, 64)], idxs_v)
            pltpu.sync_copy(dst_ref.at[pl.ds(base, 64)], idxd_v)
            cp1 = pltpu.async_copy(t1_ref.at[idxs_v], t1s, sem1)
            cp2 = pltpu.async_copy(t1_ref.at[idxd_v], t1d, sem2)
            cp1.wait()
            cp2.wait()

            def sub(subi, _):
                rofs = subi * 16
                dstv, valid, dstl = lane_ctx(base, rofs)
                row = iota + rofs
                acc_s = zeros16
                acc_v = zeros16
                for k in range(64):
                    kv = jnp.full((16,), k, _I32)
                    acc_s = acc_s + (plsc.load_gather(t1s, [row, kv]) *
                                     plsc.load_gather(t1d, [row, kv]))
                for k in range(64, 80):
                    kv = jnp.full((16,), k, _I32)
                    acc_v = acc_v + (plsc.load_gather(t1s, [row, kv]) *
                                     plsc.load_gather(t1d, [row, kv]))
                x_s = jnp.where(acc_s >= 0, acc_s, 0.2 * acc_s)
                x_v = jnp.where(acc_v >= 0, acc_v, 0.2 * acc_v)
                plsc.store_scatter(rawbuf, [(rofs + iota) * 2], x_s)
                plsc.store_scatter(rawbuf, [(rofs + iota) * 2 + 1], x_v)
                masks, run_end = seg_masks(dstv)
                wm = run_end & valid
                m = seg_scan(x_s, masks, jnp.maximum)
                cur = plsc.load_gather(mx_s, [dstl])
                plsc.store_scatter(mx_s, [dstl], jnp.maximum(cur, m), mask=wm)
                m = seg_scan(x_v, masks, jnp.maximum)
                cur = plsc.load_gather(mx_v, [dstl])
                plsc.store_scatter(mx_v, [dstl], jnp.maximum(cur, m), mask=wm)
                return 0
            lax.fori_loop(0, 4, sub, 0)
            pltpu.sync_copy(rawbuf, raw_ref.at[pl.ds(base * 2, 128)])
            return 0
        lax.fori_loop(sb0, sb1, passA, 0)

        def passB(sb, _):
            base = sb * 64
            pltpu.sync_copy(dst_ref.at[pl.ds(base, 64)], idxd_v)
            pltpu.sync_copy(raw_ref.at[pl.ds(base * 2, 128)], rawbuf)

            def sub(subi, _):
                rofs = subi * 16
                dstv, valid, dstl = lane_ctx(base, rofs)
                x_s = plsc.load_gather(rawbuf, [(rofs + iota) * 2])
                x_v = plsc.load_gather(rawbuf, [(rofs + iota) * 2 + 1])
                w_s = jnp.exp(x_s - plsc.load_gather(mx_s, [dstl]))
                w_v = jnp.exp(x_v - plsc.load_gather(mx_v, [dstl]))
                masks, run_end = seg_masks(dstv)
                wm = run_end & valid
                ssum = seg_scan(w_s, masks, lambda a, b: a + b)
                cur = plsc.load_gather(den_s, [dstl])
                plsc.store_scatter(den_s, [dstl], cur + ssum, mask=wm)
                ssum = seg_scan(w_v, masks, lambda a, b: a + b)
                cur = plsc.load_gather(den_v, [dstl])
                plsc.store_scatter(den_v, [dstl], cur + ssum, mask=wm)
                return 0
            lax.fori_loop(0, 4, sub, 0)
            return 0
        lax.fori_loop(sb0, sb1, passB, 0)

        def passC(sb, _):
            base = sb * 64
            pltpu.sync_copy(src_ref.at[pl.ds(base, 64)], idxs_v)
            pltpu.sync_copy(dst_ref.at[pl.ds(base, 64)], idxd_v)
            pltpu.sync_copy(raw_ref.at[pl.ds(base * 2, 128)], rawbuf)
            cp = pltpu.async_copy(t2_ref.at[idxs_v], t2b, sem1)
            cp.wait()

            def sub(subi, _):
                rofs = subi * 16
                dstv, valid, dstl = lane_ctx(base, rofs)
                x_s = plsc.load_gather(rawbuf, [(rofs + iota) * 2])
                x_v = plsc.load_gather(rawbuf, [(rofs + iota) * 2 + 1])
                wgt_s = (jnp.exp(x_s - plsc.load_gather(mx_s, [dstl])) /
                         (plsc.load_gather(den_s, [dstl]) + 1e-16))
                wgt_v = (jnp.exp(x_v - plsc.load_gather(mx_v, [dstl])) /
                         (plsc.load_gather(den_v, [dstl]) + 1e-16))
                valid_i = jnp.where(valid, 1, 0).astype(_I32)
                for e in range(16):
                    we_s = wgt_s[e]
                    we_v = wgt_v[e]
                    dl = dstl[e]
                    row = rofs + e

                    @pl.when(valid_i[e] == 1)
                    def _():
                        gchunk = t2b[row, pl.ds(128, 16)]
                        d = jnp.where(iota < 8,
                                      jnp.maximum(1e-4 - we_v * gchunk, 0.0),
                                      0.0)
                        need = jnp.max(d) > 0

                        @pl.when(jnp.logical_not(need))
                        def _():
                            for j in range(4):
                                pj = t2b[row, pl.ds(j * 16, 16)]
                                qj = t2b[row, pl.ds(64 + j * 16, 16)]
                                mj = jnp.maximum(we_s * pj + we_v * qj + bs_c[j], 0.0)
                                ob = dl * 64 + j * 16
                                sa_loc[pl.ds(ob, 16)] = sa_loc[pl.ds(ob, 16)] + mj

                        @pl.when(need)
                        def _():
                            for j in range(4):
                                pj = t2b[row, pl.ds(j * 16, 16)]
                                qj = t2b[row, pl.ds(64 + j * 16, 16)]
                                acc = we_s * pj + we_v * qj + bs_c[j]
                                for hh in range(8):
                                    acc = acc + d[hh] * wsv_c[hh][j]
                                ob = dl * 64 + j * 16
                                sa_loc[pl.ds(ob, 16)] = (sa_loc[pl.ds(ob, 16)] +
                                                         jnp.maximum(acc, 0.0))

                        wch = t2b[row, pl.ds(136, 16)]
                        rpch = t2b[row, pl.ds(152, 16)]
                        z = jnp.maximum(we_v * rpch, 1e-4)
                        sig = 1.0 / (1.0 + jnp.exp(-z))
                        vm = (we_v * sig) * wch
                        vb = dl * 16
                        va_loc[pl.ds(vb, 16)] = va_loc[pl.ds(vb, 16)] + vm
                return 0
            lax.fori_loop(0, 4, sub, 0)
            return 0
        lax.fori_loop(sb0, sb1, passC, 0)

        pltpu.sync_copy(sa_loc, sa_ref.at[pl.ds(n0 * 64, NN * 64)])
        pltpu.sync_copy(va_loc, va_ref.at[pl.ds(n0 * 16, NN * 16)])
        return 0

    lax.fori_loop(0, 2, vw_body, 0)


@jax.jit
def _edge_call(srcp, dstp, bnd, t1, t2, bs, wsv):
    kern = pl.kernel(
        _edge_body,
        out_type=[jax.ShapeDtypeStruct((N_PAD * 64,), _F32),
                  jax.ShapeDtypeStruct((N_PAD * 16,), _F32),
                  jax.ShapeDtypeStruct((E_PAD * 2,), _F32)],
        mesh=plsc.VectorSubcoreMesh(core_axis_name="c", subcore_axis_name="s"),
        compiler_params=pltpu.CompilerParams(needs_layout_passes=False,
                                             use_tc_tiling_on_sc=False),
        scratch_types=[
            pltpu.VMEM((80,), _I32),
            pltpu.VMEM((64,), _F32),
            pltpu.VMEM((512,), _F32),
            pltpu.VMEM((NN * 64,), _F32),
            pltpu.VMEM((NN * 16,), _F32),
            pltpu.VMEM((784,), _F32),
            pltpu.VMEM((784,), _F32),
            pltpu.VMEM((784,), _F32),
            pltpu.VMEM((784,), _F32),
            pltpu.VMEM((64,), _I32),
            pltpu.VMEM((64,), _I32),
            pltpu.VMEM((64, 80), _F32),
            pltpu.VMEM((64, 80), _F32),
            pltpu.VMEM((64, 176), _F32),
            pltpu.VMEM((128,), _F32),
            pltpu.VMEM((48,), _F32),
            pltpu.VMEM((48,), _I32),
            pltpu.SemaphoreType.DMA,
            pltpu.SemaphoreType.DMA,
        ],
    )
    return kern(srcp, dstp, bnd, t1, t2, bs, wsv)


def _flood_layer_fast(p, srcp, dstp, bnd, s, v, n):
    ne, mg = p['n_encode'], p['m_gvp']
    s_enc, v_enc = _gvp(ne, s, v)
    vh = jnp.swapaxes(v_enc, -1, -2) @ mg['wh']
    g = jnp.sqrt(jnp.sum(vh * vh, axis=-2))
    p_t = s_enc @ mg['ws'][:64]
    q_t = g @ mg['ws'][64:]
    w_t = jnp.swapaxes(vh @ mg['wv'], -1, -2)
    r_t = jnp.sqrt(jnp.sum(w_t * w_t, axis=-1))
    t1 = jnp.concatenate([s_enc, v_enc.reshape(n, 16)], axis=1)
    t2 = jnp.concatenate([p_t, q_t, g, w_t.reshape(n, 16),
                          jnp.repeat(r_t, 2, axis=1),
                          jnp.zeros((n, 8), _F32)], axis=1)
    t1 = jnp.pad(t1, ((0, N_PAD - n), (0, 0)))
    t2 = jnp.pad(t2, ((0, N_PAD - n), (0, 0)))
    sa_flat, va_flat, _ = _edge_call(srcp, dstp, bnd, t1, t2,
                                     mg['bs'], mg['ws'][64:].reshape(-1))
    s_a = sa_flat.reshape(N_PAD, 64)[:n]
    v_a = va_flat.reshape(N_PAD, 16)[:n].reshape(n, 8, 2)
    return _gvp(p['u_gvp'], s_a, v_a)


def _flood_block(p, ep, s, v, s_hid, v_hid, valid, n):
    srcp, dstp, bnd = ep
    s, v = _gvp(p['gvp_layer'], s, v, False, False)
    s_gz, v_gz = _flood_layer_fast(p['z_in'], srcp, dstp, bnd, s, v, n)
    s_gr, v_gr = _flood_layer_fast(p['r_in'], srcp, dstp, bnd, s, v, n)
    if s_hid is not None:
        a, b = _flood_layer_fast(p['z_h'], srcp, dstp, bnd, s_hid, v_hid, n)
        s_gz, v_gz = s_gz + a, v_gz + b
        a, b = _flood_layer_fast(p['r_h'], srcp, dstp, bnd, s_hid, v_hid, n)
        s_gr, v_gr = s_gr + a, v_gr + b
    s_gz, v_gz = jax.nn.sigmoid(s_gz), jax.nn.sigmoid(v_gz)
    s_gr, v_gr = jax.nn.sigmoid(s_gr), jax.nn.sigmoid(v_gr)
    s_hh, v_hh = _flood_layer_fast(p['hh_in'], srcp, dstp, bnd, s, v, n)
    if s_hid is not None:
        a, b = _flood_layer_fast(p['hh_h'], srcp, dstp, bnd,
                                 s_gr * s_hid, v_gr * v_hid, n)
        s_hh, v_hh = s_hh + a, v_hh + b
    s_hh, v_hh = jnp.tanh(s_hh), jnp.tanh(v_hh)
    s_t, v_t = (1 - s_gz) * s_hh, (1 - v_gz) * v_hh
    if s_hid is not None:
        s_t, v_t = s_gz * s_hid + s_t, v_gz * v_hid + v_t
    mask = valid[:, None]
    s_t = jnp.where(mask, s_t, 0.0)
    v_t = jnp.where(mask[..., None], v_t, 0.0)
    return s_t, v_t


def _feat_pred(p, s, v):
    s, v = _gvp(p['gvp1'], s, v)
    l_h, v_out = _gvp(p['gvp2'], s, v, False, False)
    v_old = v_out
    l_h = l_h ** 2
    s_out = _norm_no_nan(v_out, axis=-1)
    v_out = v_out / s_out[..., None]
    return l_h, (s_out, v_out), v_old


def _compute_regression(p, s, v, valid):
    mask = valid[:, None]
    l_h, (s_h, v_h), v_old = _feat_pred(p, s, v)
    l_h = jnp.where(mask, l_h, 0.0)
    s_h = jnp.where(mask, s_h, 0.0)
    v_h = jnp.where(mask[..., None], v_h, 0.0)
    return l_h, (s_h, v_h), v_old


def kernel(edge_index, s_static, seq, binary, x_v, x_v_norm, rain, wdfp, batch, params):
    n = s_static.shape[0]
    loops = jnp.arange(n, dtype=edge_index.dtype)
    src_full = jnp.concatenate([edge_index[0], loops])
    dst_full = jnp.concatenate([edge_index[1], loops])
    order = jnp.argsort(dst_full)
    dst_sorted = dst_full[order]
    srcp = jnp.pad(src_full[order], (0, E_PAD - E_TOT)).astype(_I32)
    dstp = jnp.pad(dst_sorted, (0, E_PAD - E_TOT)).astype(_I32)
    bnd = jnp.searchsorted(dst_sorted, jnp.arange(NVW + 1, dtype=_I32) * NN)
    bnd = jnp.pad(bnd.astype(_I32), (0, 80 - (NVW + 1)),
                  constant_values=E_TOT)
    ep = (srcp, dstp, bnd)

    bmask = binary.astype(bool)
    v = jnp.where(bmask[..., None], x_v, 0.0)
    v_norm = jnp.where(bmask, x_v_norm, 0.0)
    s_h, v_h = v_norm[:, 0], v[:, 0]
    l_h = wdfp[:, 0]
    r_h = rain[:, 0]
    seq_len = wdfp.shape[1] - 1
    s_h0, v_h0 = None, None
    label_loss = jnp.zeros((), _F32)
    feat_loss = jnp.zeros((), _F32)
    out_labels = []
    pp = params['processor']
    fp = params['feat_pred']
    for i in range(1, seq_len + 1):
        s_in = jnp.concatenate([s_static, s_h, r_h, l_h], axis=-1)
        valid = seq[:, i - 1]
        s_h0, v_h0 = _flood_block(pp, ep, s_in, v_h, s_h0, v_h0, valid, n)
        l_h, (s_h, v_h), v_old = _compute_regression(fp, s_h0, v_h0, valid)
        b_t = binary[:, i]
        use_t = jnp.logical_not(jnp.logical_or(jnp.all(b_t == 0), jnp.all(b_t == 1)))
        m = valid.astype(_F32)[:, None]
        label_loss = label_loss + jnp.where(
            use_t,
            jnp.sum(jnp.abs(l_h.reshape(n, -1) - wdfp[:, i].reshape(n, -1)) * m),
            jnp.zeros((), _F32))
        targ = (v[:, i] * v_norm[:, i][..., None]).reshape(n, -1)
        feat_loss = feat_loss + jnp.where(
            use_t,
            jnp.sum(jnp.abs(v_old.reshape(n, -1) - targ) * m),
            jnp.zeros((), _F32))
        out_labels.append(l_h)
        r_h = rain[:, i]
    loss = label_loss + feat_loss
    out_labels = jnp.stack(out_labels, axis=1).squeeze(-1)
    return out_labels, loss, seq
